# Initial kernel scaffold; baseline (speedup 1.0000x reference)
import jax
import jax.numpy as jnp
from jax.experimental import pallas as pl


def kernel(x, edge_index, W1_1, b1_1, W1_2, b1_2, Wm1, bm1, Wm2, bm2, Wo1, bo1, Wo2, bo2):
    raise NotImplementedError("write your pallas kernel here")


# speedup vs baseline: 13.3387x; 13.3387x over previous
import functools

import jax
import jax.numpy as jnp
from jax import lax
from jax.experimental import pallas as pl
from jax.experimental.pallas import tpu as pltpu
from jax.experimental.pallas import tpu_sc as plsc

N = 10000
NP = 10240
E = 320000
D = 128
CH = 128
NCH = E // CH
NW = 32
CHUNKS_BASE = NCH // NW
CHUNKS_REM = NCH - CHUNKS_BASE * NW
RPT = NP // 16
BM = 1024


def _scat_body(gp_hbm, src_hbm, dst_hbm, out_hbm, rowbuf, sidx, didx, acc):
    c = lax.axis_index("c")
    s = lax.axis_index("s")
    wid = s * 2 + c
    base = s * RPT
    for k in range(RPT // CH):
        pltpu.sync_copy(gp_hbm.at[pl.ds(base + k * CH, CH)], rowbuf)
        pltpu.sync_copy(rowbuf, acc.at[pl.ds(base + k * CH, CH)])
    plsc.subcore_barrier()

    start = wid * CHUNKS_BASE + jnp.minimum(wid, CHUNKS_REM)
    count = CHUNKS_BASE + jnp.where(wid < CHUNKS_REM, 1, 0)

    def chunk(j, carry):
        pltpu.sync_copy(src_hbm.at[pl.ds(j * CH, CH)], sidx)
        pltpu.sync_copy(gp_hbm.at[sidx], rowbuf)
        pltpu.sync_copy(dst_hbm.at[j], didx)
        pltpu.sync_copy(rowbuf, acc.at[didx], add=True)
        return carry

    lax.fori_loop(start, start + count, chunk, 0)
    plsc.subcore_barrier()
    for k in range(RPT // CH):
        pltpu.sync_copy(acc.at[pl.ds(base + k * CH, CH)], rowbuf)
        pltpu.sync_copy(rowbuf, out_hbm.at[c, pl.ds(base + k * CH, CH)])


def _deg_body(dst_hbm, out_hbm, vbuf, ones, didx, acc):
    c = lax.axis_index("c")
    s = lax.axis_index("s")
    wid = s * 2 + c
    for i in range(RPT // 16):
        vbuf[pl.ds(i * 16, 16)] = jnp.zeros((16,), jnp.float32)
    for i in range(CH // 16):
        ones[pl.ds(i * 16, 16)] = jnp.full((16,), 1.0, jnp.float32)
    pltpu.sync_copy(vbuf, acc.at[pl.ds(s * RPT, RPT)])
    plsc.subcore_barrier()

    start = wid * CHUNKS_BASE + jnp.minimum(wid, CHUNKS_REM)
    count = CHUNKS_BASE + jnp.where(wid < CHUNKS_REM, 1, 0)

    def chunk(j, carry):
        pltpu.sync_copy(dst_hbm.at[j], didx)
        pltpu.sync_copy(ones, acc.at[didx], add=True)
        return carry

    lax.fori_loop(start, start + count, chunk, 0)
    plsc.subcore_barrier()
    pltpu.sync_copy(acc.at[pl.ds(s * RPT, RPT)], vbuf)
    pltpu.sync_copy(vbuf, out_hbm.at[c, pl.ds(s * RPT, RPT)])


@functools.lru_cache(maxsize=None)
def _sc_kernels():
    mesh = plsc.VectorSubcoreMesh(core_axis_name="c", subcore_axis_name="s")
    scat = pl.kernel(
        _scat_body,
        out_type=jax.ShapeDtypeStruct((2, NP, D), jnp.float32),
        mesh=mesh,
        scratch_types=[
            pltpu.VMEM((CH, D), jnp.float32),
            pltpu.VMEM((CH,), jnp.int32),
            pltpu.VMEM((CH,), jnp.int32),
            pltpu.VMEM_SHARED((NP, D), jnp.float32),
        ],
    )
    deg = pl.kernel(
        _deg_body,
        out_type=jax.ShapeDtypeStruct((2, NP), jnp.float32),
        mesh=mesh,
        scratch_types=[
            pltpu.VMEM((RPT,), jnp.float32),
            pltpu.VMEM((CH,), jnp.float32),
            pltpu.VMEM((CH,), jnp.int32),
            pltpu.VMEM_SHARED((NP,), jnp.float32),
        ],
    )
    return scat, deg


def _tc_pre_body(x_ref, d0_ref, d1_ref, W1_ref, b1_ref, W2_ref,
                 dense_ref, gp_ref, dinv_ref):
    deg = d0_ref[...] + d1_ref[...] + 1.0
    dinv = lax.rsqrt(deg)
    xb = x_ref[...]
    dense_ref[...] = (
        jnp.dot(xb, W1_ref[...], preferred_element_type=jnp.float32)
        + b1_ref[...]
    )
    gp_ref[...] = dinv[:, None] * jnp.dot(
        xb, W2_ref[...], preferred_element_type=jnp.float32)
    dinv_ref[...] = dinv


def _tc_layer_body(with_h, dense_ref, gp_ref, parts_ref, dinv_ref, bc_ref,
                   Wd_ref, bd_ref, Wg_ref, *out_refs):
    dinv = dinv_ref[...]
    gp = gp_ref[...]
    accs = parts_ref[0] + parts_ref[1] - gp
    conv = dinv[:, None] * accs + bc_ref[...]
    h = jnp.maximum(dense_ref[...] + conv, 0.0)
    mean = jnp.mean(h, axis=1, keepdims=True)
    dlt = h - mean
    var = jnp.sum(dlt * dlt, axis=1, keepdims=True) * (1.0 / (D - 1))
    hn = dlt / (jnp.sqrt(var) + 1e-5)
    out_refs[0][...] = (
        jnp.dot(hn, Wd_ref[...], preferred_element_type=jnp.float32)
        + bd_ref[...]
    )
    out_refs[1][...] = dinv[:, None] * jnp.dot(
        hn, Wg_ref[...], preferred_element_type=jnp.float32)
    if with_h:
        out_refs[2][...] = hn


def _tc_fin_body(dense_ref, gp_ref, parts_ref, dinv_ref, bc_ref, out_ref):
    accs = parts_ref[0] + parts_ref[1] - gp_ref[...]
    out_ref[...] = dense_ref[...] + dinv_ref[...][:, None] * accs + bc_ref[...]


_ROWS = pl.BlockSpec((BM, D), lambda i: (i, 0))
_VECN = pl.BlockSpec((BM,), lambda i: (i,))
_WFULL = pl.BlockSpec((D, D), lambda i: (0, 0))
_BFULL = pl.BlockSpec((D,), lambda i: (0,))
_PARTS = pl.BlockSpec((2, BM, D), lambda i: (0, i, 0))
_GRID = (NP // BM,)

_mat = jax.ShapeDtypeStruct((NP, D), jnp.float32)
_vec = jax.ShapeDtypeStruct((NP,), jnp.float32)

_tc_pre = pl.pallas_call(
    _tc_pre_body,
    grid=_GRID,
    in_specs=[_ROWS, _VECN, _VECN, _WFULL, _BFULL, _WFULL],
    out_specs=[_ROWS, _ROWS, _VECN],
    out_shape=[_mat, _mat, _vec],
)

_tc_mid = pl.pallas_call(
    functools.partial(_tc_layer_body, False),
    grid=_GRID,
    in_specs=[_ROWS, _ROWS, _PARTS, _VECN, _BFULL, _WFULL, _BFULL, _WFULL],
    out_specs=[_ROWS, _ROWS],
    out_shape=[_mat, _mat],
)

_tc_out = pl.pallas_call(
    functools.partial(_tc_layer_body, True),
    grid=_GRID,
    in_specs=[_ROWS, _ROWS, _PARTS, _VECN, _BFULL, _WFULL, _BFULL, _WFULL],
    out_specs=[_ROWS, _ROWS, _ROWS],
    out_shape=[_mat, _mat, _mat],
)

_tc_fin = pl.pallas_call(
    _tc_fin_body,
    grid=_GRID,
    in_specs=[_ROWS, _ROWS, _PARTS, _VECN, _BFULL],
    out_specs=_ROWS,
    out_shape=_mat,
)


def kernel(x, edge_index, W1_1, b1_1, W1_2, b1_2, Wm1, bm1, Wm2, bm2,
           Wo1, bo1, Wo2, bo2):
    src = edge_index[0]
    dst2 = edge_index[1].reshape(NCH, CH)
    x_p = jnp.pad(x, ((0, NP - N), (0, 0)))

    _scat, _deg = _sc_kernels()
    degp = _deg(dst2)
    dense1, gp1, dinv = _tc_pre(x_p, degp[0], degp[1], W1_1, b1_1, W1_2)
    parts1 = _scat(gp1, src, dst2)
    dense2, gp2 = _tc_mid(dense1, gp1, parts1, dinv, b1_2, Wm1, bm1, Wm2)
    parts2 = _scat(gp2, src, dst2)
    dense3, gp3, pen = _tc_out(dense2, gp2, parts2, dinv, bm2, Wo1, bo1, Wo2)
    parts3 = _scat(gp3, src, dst2)
    out = _tc_fin(dense3, gp3, parts3, dinv, bo2)
    return out[:N], pen[:N]


# speedup vs baseline: 24.2979x; 1.8216x over previous
import functools

import jax
import jax.numpy as jnp
from jax import lax
from jax.experimental import pallas as pl
from jax.experimental.pallas import tpu as pltpu
from jax.experimental.pallas import tpu_sc as plsc

N = 10000
NP = 10240
E = 320000
D = 128
CH = 128
NW = 32
CPT = 80
NCHP = NW * CPT
EP = NCHP * CH
NB = 4
RPT = NP // 16
BM = 1024


def _scat_body(gp_hbm, src_hbm, dst_hbm, out_hbm,
               r0, r1, s0, s1, d0, d1, acc, gs0, gs1, is0, is1):
    rb = [r0, r1]
    sx = [s0, s1]
    dx = [d0, d1]
    gsem = [gs0, gs1]
    isem = [is0, is1]
    c = lax.axis_index("c")
    s = lax.axis_index("s")
    wid = s * 2 + c
    base = s * RPT
    cbase = wid * CPT

    def fetch_idx(j, b):
        pltpu.async_copy(src_hbm.at[cbase + j], sx[b], isem[b])
        pltpu.async_copy(dst_hbm.at[cbase + j], dx[b], isem[b])

    def wait_idx(b):
        pltpu.make_async_copy(src_hbm.at[0], sx[b], isem[b]).wait()
        pltpu.make_async_copy(dst_hbm.at[0], dx[b], isem[b]).wait()

    def wait_gather(b):
        pltpu.make_async_copy(gp_hbm.at[sx[b]], rb[b], gsem[b]).wait()

    fetch_idx(0, 0)
    fetch_idx(1, 1)
    for k in range(RPT // CH):
        pltpu.sync_copy(gp_hbm.at[pl.ds(base + k * CH, CH)], rb[k % 2])
        pltpu.sync_copy(rb[k % 2], acc.at[pl.ds(base + k * CH, CH)])
    wait_idx(0)
    plsc.subcore_barrier()
    pltpu.async_copy(gp_hbm.at[sx[0]], rb[0], gsem[0])

    def rnd(t, carry):
        for b in range(2):
            j = 2 * t + b
            nb = 1 - b
            @pl.when(j + 1 < CPT)
            def _():
                wait_idx(nb)
                pltpu.async_copy(gp_hbm.at[sx[nb]], rb[nb], gsem[nb])
            wait_gather(b)
            pltpu.sync_copy(rb[b], acc.at[dx[b]], add=True)
            @pl.when(j + 2 < CPT)
            def _():
                fetch_idx(j + 2, b)
        return carry

    lax.fori_loop(0, CPT // 2, rnd, 0)
    plsc.subcore_barrier()
    for k in range(RPT // CH):
        pltpu.sync_copy(acc.at[pl.ds(base + k * CH, CH)], rb[k % 2])
        pltpu.sync_copy(rb[k % 2], out_hbm.at[c, pl.ds(base + k * CH, CH)])


def _deg_body(dst_hbm, out_hbm, vbuf, ones, dbuf, acc, dsem, isem):
    c = lax.axis_index("c")
    s = lax.axis_index("s")
    wid = s * 2 + c
    ic = pltpu.async_copy(dst_hbm.at[pl.ds(wid * CPT, CPT)], dbuf, isem)
    for i in range(RPT // 16):
        vbuf[pl.ds(i * 16, 16)] = jnp.zeros((16,), jnp.float32)
    for i in range(CH // 16):
        ones[pl.ds(i * 16, 16)] = jnp.full((16,), 1.0, jnp.float32)
    pltpu.sync_copy(vbuf, acc.at[pl.ds(s * RPT, RPT)])
    ic.wait()
    plsc.subcore_barrier()

    def rnd(t, carry):
        for b in range(NB):
            pltpu.async_copy(ones, acc.at[dbuf.at[t * NB + b]], dsem, add=True)
        for b in range(NB):
            pltpu.make_async_copy(ones, acc.at[dbuf.at[0]], dsem).wait()
        return carry

    lax.fori_loop(0, CPT // NB, rnd, 0)
    plsc.subcore_barrier()
    pltpu.sync_copy(acc.at[pl.ds(s * RPT, RPT)], vbuf)
    pltpu.sync_copy(vbuf, out_hbm.at[c, pl.ds(s * RPT, RPT)])


@functools.lru_cache(maxsize=None)
def _sc_kernels():
    mesh = plsc.VectorSubcoreMesh(core_axis_name="c", subcore_axis_name="s")
    scat = pl.kernel(
        _scat_body,
        out_type=jax.ShapeDtypeStruct((2, NP, D), jnp.float32),
        mesh=mesh,
        scratch_types=(
            [pltpu.VMEM((CH, D), jnp.float32) for _ in range(2)]
            + [pltpu.VMEM((CH,), jnp.int32) for _ in range(4)]
            + [pltpu.VMEM_SHARED((NP, D), jnp.float32)]
            + [pltpu.SemaphoreType.DMA] * 4
        ),
    )
    deg = pl.kernel(
        _deg_body,
        out_type=jax.ShapeDtypeStruct((2, NP), jnp.float32),
        mesh=mesh,
        scratch_types=(
            [pltpu.VMEM((RPT,), jnp.float32),
             pltpu.VMEM((CH,), jnp.float32),
             pltpu.VMEM((CPT, CH), jnp.int32),
             pltpu.VMEM_SHARED((NP,), jnp.float32)]
            + [pltpu.SemaphoreType.DMA] * 2
        ),
    )
    return scat, deg


def _tc_pre_body(x_ref, d0_ref, d1_ref, W1_ref, b1_ref, W2_ref,
                 dense_ref, gp_ref, dinv_ref):
    deg = d0_ref[...] + d1_ref[...] + 1.0
    dinv = lax.rsqrt(deg)
    xb = x_ref[...]
    dense_ref[...] = (
        jnp.dot(xb, W1_ref[...], preferred_element_type=jnp.float32)
        + b1_ref[...]
    )
    gp_ref[...] = dinv[:, None] * jnp.dot(
        xb, W2_ref[...], preferred_element_type=jnp.float32)
    dinv_ref[...] = dinv


def _tc_layer_body(with_h, dense_ref, gp_ref, parts_ref, dinv_ref, bc_ref,
                   Wd_ref, bd_ref, Wg_ref, *out_refs):
    dinv = dinv_ref[...]
    gp = gp_ref[...]
    accs = parts_ref[0] + parts_ref[1] - gp
    conv = dinv[:, None] * accs + bc_ref[...]
    h = jnp.maximum(dense_ref[...] + conv, 0.0)
    mean = jnp.mean(h, axis=1, keepdims=True)
    dlt = h - mean
    var = jnp.sum(dlt * dlt, axis=1, keepdims=True) * (1.0 / (D - 1))
    hn = dlt / (jnp.sqrt(var) + 1e-5)
    out_refs[0][...] = (
        jnp.dot(hn, Wd_ref[...], preferred_element_type=jnp.float32)
        + bd_ref[...]
    )
    out_refs[1][...] = dinv[:, None] * jnp.dot(
        hn, Wg_ref[...], preferred_element_type=jnp.float32)
    if with_h:
        out_refs[2][...] = hn


def _tc_fin_body(dense_ref, gp_ref, parts_ref, dinv_ref, bc_ref, out_ref):
    accs = parts_ref[0] + parts_ref[1] - gp_ref[...]
    out_ref[...] = dense_ref[...] + dinv_ref[...][:, None] * accs + bc_ref[...]


_ROWS = pl.BlockSpec((BM, D), lambda i: (i, 0))
_VECN = pl.BlockSpec((BM,), lambda i: (i,))
_WFULL = pl.BlockSpec((D, D), lambda i: (0, 0))
_BFULL = pl.BlockSpec((D,), lambda i: (0,))
_PARTS = pl.BlockSpec((2, BM, D), lambda i: (0, i, 0))
_GRID = (NP // BM,)

_mat = jax.ShapeDtypeStruct((NP, D), jnp.float32)
_vec = jax.ShapeDtypeStruct((NP,), jnp.float32)

_tc_pre = pl.pallas_call(
    _tc_pre_body,
    grid=_GRID,
    in_specs=[_ROWS, _VECN, _VECN, _WFULL, _BFULL, _WFULL],
    out_specs=[_ROWS, _ROWS, _VECN],
    out_shape=[_mat, _mat, _vec],
)

_tc_mid = pl.pallas_call(
    functools.partial(_tc_layer_body, False),
    grid=_GRID,
    in_specs=[_ROWS, _ROWS, _PARTS, _VECN, _BFULL, _WFULL, _BFULL, _WFULL],
    out_specs=[_ROWS, _ROWS],
    out_shape=[_mat, _mat],
)

_tc_out = pl.pallas_call(
    functools.partial(_tc_layer_body, True),
    grid=_GRID,
    in_specs=[_ROWS, _ROWS, _PARTS, _VECN, _BFULL, _WFULL, _BFULL, _WFULL],
    out_specs=[_ROWS, _ROWS, _ROWS],
    out_shape=[_mat, _mat, _mat],
)

_tc_fin = pl.pallas_call(
    _tc_fin_body,
    grid=_GRID,
    in_specs=[_ROWS, _ROWS, _PARTS, _VECN, _BFULL],
    out_specs=_ROWS,
    out_shape=_mat,
)


def kernel(x, edge_index, W1_1, b1_1, W1_2, b1_2, Wm1, bm1, Wm2, bm2,
           Wo1, bo1, Wo2, bo2):
    pad = jnp.arange(EP - E, dtype=jnp.int32)
    src2 = jnp.concatenate([edge_index[0], pad % NP]).reshape(NCHP, CH)
    dst2 = jnp.concatenate([edge_index[1], N + pad % (NP - N)]).reshape(NCHP, CH)
    x_p = jnp.pad(x, ((0, NP - N), (0, 0)))

    _scat, _deg = _sc_kernels()
    degp = _deg(dst2)
    dense1, gp1, dinv = _tc_pre(x_p, degp[0], degp[1], W1_1, b1_1, W1_2)
    parts1 = _scat(gp1, src2, dst2)
    dense2, gp2 = _tc_mid(dense1, gp1, parts1, dinv, b1_2, Wm1, bm1, Wm2)
    parts2 = _scat(gp2, src2, dst2)
    dense3, gp3, pen = _tc_out(dense2, gp2, parts2, dinv, bm2, Wo1, bo1, Wo2)
    parts3 = _scat(gp3, src2, dst2)
    out = _tc_fin(dense3, gp3, parts3, dinv, bo2)
    return out[:N], pen[:N]


# speedup vs baseline: 27.0849x; 1.1147x over previous
import functools

import jax
import jax.numpy as jnp
from jax import lax
from jax.experimental import pallas as pl
from jax.experimental.pallas import tpu as pltpu
from jax.experimental.pallas import tpu_sc as plsc

N = 10000
NP = 10240
E = 320000
D = 128
CH = 128
NW = 32
CPT = 80
NCHP = NW * CPT
EP = NCHP * CH
NB = 4
RPT = NP // 16
BM = 1024


def _scat_body(gp_hbm, src_hbm, dst_hbm, out_hbm,
               r0, r1, s0, s1, s2, s3, d0, d1, d2, d3, acc,
               gs0, gs1, is0, is1, is2, is3, ss0, ss1):
    rb = [r0, r1]
    sx = [s0, s1, s2, s3]
    dx = [d0, d1, d2, d3]
    gsem = [gs0, gs1]
    isem = [is0, is1, is2, is3]
    ssem = [ss0, ss1]
    c = lax.axis_index("c")
    s = lax.axis_index("s")
    wid = s * 2 + c
    base = s * RPT
    cbase = wid * CPT

    def fetch_idx(j, q):
        pltpu.async_copy(src_hbm.at[cbase + j], sx[q], isem[q])
        pltpu.async_copy(dst_hbm.at[cbase + j], dx[q], isem[q])

    def wait_idx(q):
        pltpu.make_async_copy(src_hbm.at[0], sx[q], isem[q]).wait()
        pltpu.make_async_copy(dst_hbm.at[0], dx[q], isem[q]).wait()

    def wait_gather(b):
        pltpu.make_async_copy(gp_hbm.at[sx[0]], rb[b], gsem[b]).wait()

    def wait_scat(b):
        pltpu.make_async_copy(gp_hbm.at[pl.ds(0, CH)], rb[b], ssem[b]).wait()

    fetch_idx(0, 0)
    fetch_idx(1, 1)
    for k in range(RPT // CH):
        pltpu.sync_copy(gp_hbm.at[pl.ds(base + k * CH, CH)], rb[k % 2])
        pltpu.sync_copy(rb[k % 2], acc.at[pl.ds(base + k * CH, CH)])
    wait_idx(0)
    plsc.subcore_barrier()
    pltpu.async_copy(gp_hbm.at[sx[0]], rb[0], gsem[0])

    def rnd(t, carry):
        for k in range(4):
            j = 4 * t + k
            b = k % 2
            @pl.when(j > 0)
            def _():
                wait_scat(1 - b)
            @pl.when(j + 2 < CPT)
            def _():
                fetch_idx(j + 2, (k + 2) % 4)
            @pl.when(j + 1 < CPT)
            def _():
                wait_idx((k + 1) % 4)
                pltpu.async_copy(gp_hbm.at[sx[(k + 1) % 4]],
                                 rb[1 - b], gsem[1 - b])
            wait_gather(b)
            pltpu.async_copy(rb[b], acc.at[dx[k]], ssem[b], add=True)
        return carry

    lax.fori_loop(0, CPT // 4, rnd, 0)
    wait_scat((CPT - 1) % 2)
    plsc.subcore_barrier()
    for k in range(RPT // CH):
        pltpu.sync_copy(acc.at[pl.ds(base + k * CH, CH)], rb[k % 2])
        pltpu.sync_copy(rb[k % 2], out_hbm.at[c, pl.ds(base + k * CH, CH)])


def _deg_body(dst_hbm, out_hbm, vbuf, ones, dbuf, acc, dsem, isem):
    c = lax.axis_index("c")
    s = lax.axis_index("s")
    wid = s * 2 + c
    ic = pltpu.async_copy(dst_hbm.at[pl.ds(wid * CPT, CPT)], dbuf, isem)
    for i in range(RPT // 16):
        vbuf[pl.ds(i * 16, 16)] = jnp.zeros((16,), jnp.float32)
    for i in range(CH // 16):
        ones[pl.ds(i * 16, 16)] = jnp.full((16,), 1.0, jnp.float32)
    pltpu.sync_copy(vbuf, acc.at[pl.ds(s * RPT, RPT)])
    ic.wait()
    plsc.subcore_barrier()

    def rnd(t, carry):
        for b in range(NB):
            pltpu.async_copy(ones, acc.at[dbuf.at[t * NB + b]], dsem, add=True)
        for b in range(NB):
            pltpu.make_async_copy(ones, acc.at[dbuf.at[0]], dsem).wait()
        return carry

    lax.fori_loop(0, CPT // NB, rnd, 0)
    plsc.subcore_barrier()
    pltpu.sync_copy(acc.at[pl.ds(s * RPT, RPT)], vbuf)
    pltpu.sync_copy(vbuf, out_hbm.at[c, pl.ds(s * RPT, RPT)])


@functools.lru_cache(maxsize=None)
def _sc_kernels():
    mesh = plsc.VectorSubcoreMesh(core_axis_name="c", subcore_axis_name="s")
    scat = pl.kernel(
        _scat_body,
        out_type=jax.ShapeDtypeStruct((2, NP, D), jnp.float32),
        mesh=mesh,
        scratch_types=(
            [pltpu.VMEM((CH, D), jnp.float32) for _ in range(2)]
            + [pltpu.VMEM((CH,), jnp.int32) for _ in range(8)]
            + [pltpu.VMEM_SHARED((NP, D), jnp.float32)]
            + [pltpu.SemaphoreType.DMA] * 8
        ),
    )
    deg = pl.kernel(
        _deg_body,
        out_type=jax.ShapeDtypeStruct((2, NP), jnp.float32),
        mesh=mesh,
        scratch_types=(
            [pltpu.VMEM((RPT,), jnp.float32),
             pltpu.VMEM((CH,), jnp.float32),
             pltpu.VMEM((CPT, CH), jnp.int32),
             pltpu.VMEM_SHARED((NP,), jnp.float32)]
            + [pltpu.SemaphoreType.DMA] * 2
        ),
    )
    return scat, deg


def _tc_pre_body(x_ref, d0_ref, d1_ref, W1_ref, b1_ref, W2_ref,
                 dense_ref, gp_ref, dinv_ref):
    deg = d0_ref[...] + d1_ref[...] + 1.0
    dinv = lax.rsqrt(deg)
    xb = x_ref[...]
    dense_ref[...] = (
        jnp.dot(xb, W1_ref[...], preferred_element_type=jnp.float32)
        + b1_ref[...]
    )
    gp_ref[...] = dinv[:, None] * jnp.dot(
        xb, W2_ref[...], preferred_element_type=jnp.float32)
    dinv_ref[...] = dinv


def _tc_layer_body(with_h, dense_ref, gp_ref, parts_ref, dinv_ref, bc_ref,
                   Wd_ref, bd_ref, Wg_ref, *out_refs):
    dinv = dinv_ref[...]
    gp = gp_ref[...]
    accs = parts_ref[0] + parts_ref[1] - gp
    conv = dinv[:, None] * accs + bc_ref[...]
    h = jnp.maximum(dense_ref[...] + conv, 0.0)
    mean = jnp.mean(h, axis=1, keepdims=True)
    dlt = h - mean
    var = jnp.sum(dlt * dlt, axis=1, keepdims=True) * (1.0 / (D - 1))
    hn = dlt / (jnp.sqrt(var) + 1e-5)
    out_refs[0][...] = (
        jnp.dot(hn, Wd_ref[...], preferred_element_type=jnp.float32)
        + bd_ref[...]
    )
    out_refs[1][...] = dinv[:, None] * jnp.dot(
        hn, Wg_ref[...], preferred_element_type=jnp.float32)
    if with_h:
        out_refs[2][...] = hn


def _tc_fin_body(dense_ref, gp_ref, parts_ref, dinv_ref, bc_ref, out_ref):
    accs = parts_ref[0] + parts_ref[1] - gp_ref[...]
    out_ref[...] = dense_ref[...] + dinv_ref[...][:, None] * accs + bc_ref[...]


_ROWS = pl.BlockSpec((BM, D), lambda i: (i, 0))
_VECN = pl.BlockSpec((BM,), lambda i: (i,))
_WFULL = pl.BlockSpec((D, D), lambda i: (0, 0))
_BFULL = pl.BlockSpec((D,), lambda i: (0,))
_PARTS = pl.BlockSpec((2, BM, D), lambda i: (0, i, 0))
_GRID = (NP // BM,)

_mat = jax.ShapeDtypeStruct((NP, D), jnp.float32)
_vec = jax.ShapeDtypeStruct((NP,), jnp.float32)

_tc_pre = pl.pallas_call(
    _tc_pre_body,
    grid=_GRID,
    in_specs=[_ROWS, _VECN, _VECN, _WFULL, _BFULL, _WFULL],
    out_specs=[_ROWS, _ROWS, _VECN],
    out_shape=[_mat, _mat, _vec],
)

_tc_mid = pl.pallas_call(
    functools.partial(_tc_layer_body, False),
    grid=_GRID,
    in_specs=[_ROWS, _ROWS, _PARTS, _VECN, _BFULL, _WFULL, _BFULL, _WFULL],
    out_specs=[_ROWS, _ROWS],
    out_shape=[_mat, _mat],
)

_tc_out = pl.pallas_call(
    functools.partial(_tc_layer_body, True),
    grid=_GRID,
    in_specs=[_ROWS, _ROWS, _PARTS, _VECN, _BFULL, _WFULL, _BFULL, _WFULL],
    out_specs=[_ROWS, _ROWS, _ROWS],
    out_shape=[_mat, _mat, _mat],
)

_tc_fin = pl.pallas_call(
    _tc_fin_body,
    grid=_GRID,
    in_specs=[_ROWS, _ROWS, _PARTS, _VECN, _BFULL],
    out_specs=_ROWS,
    out_shape=_mat,
)


def kernel(x, edge_index, W1_1, b1_1, W1_2, b1_2, Wm1, bm1, Wm2, bm2,
           Wo1, bo1, Wo2, bo2):
    pad = jnp.arange(EP - E, dtype=jnp.int32)
    src2 = jnp.concatenate([edge_index[0], pad % NP]).reshape(NCHP, CH)
    dst2 = jnp.concatenate([edge_index[1], N + pad % (NP - N)]).reshape(NCHP, CH)
    x_p = jnp.pad(x, ((0, NP - N), (0, 0)))

    _scat, _deg = _sc_kernels()
    degp = _deg(dst2)
    dense1, gp1, dinv = _tc_pre(x_p, degp[0], degp[1], W1_1, b1_1, W1_2)
    parts1 = _scat(gp1, src2, dst2)
    dense2, gp2 = _tc_mid(dense1, gp1, parts1, dinv, b1_2, Wm1, bm1, Wm2)
    parts2 = _scat(gp2, src2, dst2)
    dense3, gp3, pen = _tc_out(dense2, gp2, parts2, dinv, bm2, Wo1, bo1, Wo2)
    parts3 = _scat(gp3, src2, dst2)
    out = _tc_fin(dense3, gp3, parts3, dinv, bo2)
    return out[:N], pen[:N]
